# Initial kernel scaffold; baseline (speedup 1.0000x reference)
import jax
import jax.numpy as jnp
from jax.experimental import pallas as pl


def kernel(x, edge_index, W1, b1, W2, b2):
    raise NotImplementedError("write your pallas kernel here")


# speedup vs baseline: 16.1795x; 16.1795x over previous
import functools

import jax
import jax.numpy as jnp
from jax import lax
from jax.experimental import pallas as pl
from jax.experimental.pallas import tpu as pltpu
from jax.experimental.pallas import tpu_sc as plsc

NC = 2
NS = 16
NW = NC * NS
K = 128


def _sc_mesh():
    return plsc.VectorSubcoreMesh(core_axis_name="c", subcore_axis_name="s")


DW = 16


@functools.lru_cache(maxsize=None)
def _make_deg(n_pad, e_w):
    zrows = n_pad // NS

    @functools.partial(
        pl.kernel,
        out_type=jax.ShapeDtypeStruct((NC * n_pad, DW), jnp.float32),
        mesh=_sc_mesh(),
        compiler_params=pltpu.CompilerParams(use_tc_tiling_on_sc=False),
        scratch_types=[
            pltpu.VMEM((K,), jnp.int32),
            pltpu.VMEM((K, DW), jnp.float32),
            pltpu.VMEM_SHARED((n_pad, DW), jnp.float32),
        ],
    )
    def deg_kernel(dst_hbm, ones_hbm, zero_hbm, out_hbm, didx, ones_v, acc):
        cid = lax.axis_index("c")
        sid = lax.axis_index("s")
        pltpu.sync_copy(zero_hbm.at[pl.ds(sid * zrows, zrows)],
                        acc.at[pl.ds(sid * zrows, zrows)])
        pltpu.sync_copy(ones_hbm, ones_v)
        plsc.subcore_barrier()
        base = (cid * NS + sid) * e_w

        def body(i, carry):
            pltpu.sync_copy(dst_hbm.at[pl.ds(base + i * K, K)], didx)
            pltpu.sync_copy(ones_v, acc.at[didx], add=True)
            return carry

        lax.fori_loop(0, e_w // K, body, 0)
        plsc.subcore_barrier()
        pltpu.sync_copy(acc.at[pl.ds(sid * zrows, zrows)],
                        out_hbm.at[pl.ds(cid * n_pad + sid * zrows, zrows)])

    return deg_kernel


@functools.lru_cache(maxsize=None)
def _make_agg(n_pad, d, e_w):
    zrows = n_pad // NS

    @functools.partial(
        pl.kernel,
        out_type=jax.ShapeDtypeStruct((NC * n_pad, d), jnp.float32),
        mesh=_sc_mesh(),
        compiler_params=pltpu.CompilerParams(use_tc_tiling_on_sc=False),
        scratch_types=[
            pltpu.VMEM((K,), jnp.int32),
            pltpu.VMEM((K,), jnp.int32),
            pltpu.VMEM((K, d), jnp.float32),
            pltpu.VMEM_SHARED((n_pad, d), jnp.float32),
            pltpu.SemaphoreType.DMA,
        ],
    )
    def agg_kernel(src_hbm, dst_hbm, h_hbm, zero_hbm, out_hbm,
                   sidx, didx, rows, acc, sem):
        cid = lax.axis_index("c")
        sid = lax.axis_index("s")
        pltpu.sync_copy(zero_hbm.at[pl.ds(sid * zrows, zrows)],
                        acc.at[pl.ds(sid * zrows, zrows)])
        plsc.subcore_barrier()
        base = (cid * NS + sid) * e_w

        def body(i, carry):
            off = base + i * K
            pltpu.sync_copy(src_hbm.at[pl.ds(off, K)], sidx)
            pltpu.sync_copy(dst_hbm.at[pl.ds(off, K)], didx)
            pltpu.async_copy(h_hbm.at[sidx], rows, sem).wait()
            pltpu.sync_copy(rows, acc.at[didx], add=True)
            return carry

        lax.fori_loop(0, e_w // K, body, 0)
        plsc.subcore_barrier()
        pltpu.sync_copy(acc.at[pl.ds(sid * zrows, zrows)],
                        out_hbm.at[pl.ds(cid * n_pad + sid * zrows, zrows)])

    return agg_kernel


def _tc1_body(x_ref, w_ref, d0_ref, d1_ref, h_ref, dis_ref):
    deg = d0_ref[...] + d1_ref[...] + 1.0
    dis = lax.rsqrt(deg)
    dis_ref[...] = dis
    h_ref[...] = dis * jnp.dot(x_ref[...], w_ref[...],
                               preferred_element_type=jnp.float32)


def _tc2_body(a0_ref, a1_ref, h1s_ref, dis_ref, b1_ref, w2_ref, out_ref):
    dis = dis_ref[...]
    h = dis * (a0_ref[...] + a1_ref[...] + h1s_ref[...]) + b1_ref[...]
    h = jnp.maximum(h, 0.0)
    out_ref[...] = dis * jnp.dot(h, w2_ref[...],
                                 preferred_element_type=jnp.float32)


def _tc3_body(a0_ref, a1_ref, h2s_ref, dis_ref, b2_ref, out_ref):
    dis = dis_ref[...]
    t = dis * (a0_ref[...] + a1_ref[...] + h2s_ref[...]) + b2_ref[...]
    m = jnp.max(t, axis=1, keepdims=True)
    lse = jnp.log(jnp.sum(jnp.exp(t - m), axis=1, keepdims=True)) + m
    out_ref[...] = t - lse


def kernel(x, edge_index, W1, b1, W2, b2):
    N, d_in = x.shape
    d_h = W1.shape[1]
    d_out = W2.shape[1]
    E = edge_index.shape[1]
    f32 = jnp.float32

    e_w = -(-E // (NW * K)) * K
    e_pad = NW * e_w
    n_pad = -(-(N + 1) // 128) * 128
    pad = e_pad - E

    src_p = jnp.concatenate([edge_index[0],
                             jnp.zeros((pad,), edge_index.dtype)])
    dst_p = jnp.concatenate([edge_index[1],
                             jnp.full((pad,), N, edge_index.dtype)])

    degs = _make_deg(n_pad, e_w)(
        dst_p, jnp.ones((K, DW), f32), jnp.zeros((n_pad, DW), f32))
    d0 = degs[:N, :1]
    d1 = degs[n_pad:n_pad + N, :1]

    R = 2000
    grid = (N // R,)
    h1s, dis = pl.pallas_call(
        _tc1_body,
        grid=grid,
        in_specs=[
            pl.BlockSpec((R, d_in), lambda i: (i, 0)),
            pl.BlockSpec((d_in, d_h), lambda i: (0, 0)),
            pl.BlockSpec((R, 1), lambda i: (i, 0)),
            pl.BlockSpec((R, 1), lambda i: (i, 0)),
        ],
        out_specs=[
            pl.BlockSpec((R, d_h), lambda i: (i, 0)),
            pl.BlockSpec((R, 1), lambda i: (i, 0)),
        ],
        out_shape=[
            jax.ShapeDtypeStruct((N, d_h), f32),
            jax.ShapeDtypeStruct((N, 1), f32),
        ],
    )(x, W1, d0, d1)

    agg1 = _make_agg(n_pad, d_h, e_w)(
        src_p, dst_p, h1s, jnp.zeros((n_pad, d_h), f32))

    h2s = pl.pallas_call(
        _tc2_body,
        grid=grid,
        in_specs=[
            pl.BlockSpec((R, d_h), lambda i: (i, 0)),
            pl.BlockSpec((R, d_h), lambda i: (i, 0)),
            pl.BlockSpec((R, d_h), lambda i: (i, 0)),
            pl.BlockSpec((R, 1), lambda i: (i, 0)),
            pl.BlockSpec((1, d_h), lambda i: (0, 0)),
            pl.BlockSpec((d_h, d_out), lambda i: (0, 0)),
        ],
        out_specs=pl.BlockSpec((R, d_out), lambda i: (i, 0)),
        out_shape=jax.ShapeDtypeStruct((N, d_out), f32),
    )(agg1[:N], agg1[n_pad:n_pad + N], h1s, dis,
      b1.reshape(1, d_h), W2)

    agg2 = _make_agg(n_pad, d_out, e_w)(
        src_p, dst_p, h2s, jnp.zeros((n_pad, d_out), f32))

    out = pl.pallas_call(
        _tc3_body,
        grid=grid,
        in_specs=[
            pl.BlockSpec((R, d_out), lambda i: (i, 0)),
            pl.BlockSpec((R, d_out), lambda i: (i, 0)),
            pl.BlockSpec((R, d_out), lambda i: (i, 0)),
            pl.BlockSpec((R, 1), lambda i: (i, 0)),
            pl.BlockSpec((1, d_out), lambda i: (0, 0)),
        ],
        out_specs=pl.BlockSpec((R, d_out), lambda i: (i, 0)),
        out_shape=jax.ShapeDtypeStruct((N, d_out), f32),
    )(agg2[:N], agg2[n_pad:n_pad + N], h2s, dis, b2.reshape(1, d_out))

    return out


# speedup vs baseline: 24.6677x; 1.5246x over previous
import functools

import jax
import jax.numpy as jnp
from jax import lax
from jax.experimental import pallas as pl
from jax.experimental.pallas import tpu as pltpu
from jax.experimental.pallas import tpu_sc as plsc

NC = 2
NS = 16
NW = NC * NS
K = 128


def _sc_mesh():
    return plsc.VectorSubcoreMesh(core_axis_name="c", subcore_axis_name="s")


DW = 16


@functools.lru_cache(maxsize=None)
def _make_deg(n_pad, e_w):
    zrows = n_pad // NS
    nc = e_w // K

    @functools.partial(
        pl.kernel,
        out_type=jax.ShapeDtypeStruct((NC * n_pad, DW), jnp.float32),
        mesh=_sc_mesh(),
        compiler_params=pltpu.CompilerParams(use_tc_tiling_on_sc=False),
        scratch_types=[
            pltpu.VMEM((nc, K), jnp.int32),
            pltpu.VMEM((K, DW), jnp.float32),
            pltpu.VMEM_SHARED((n_pad, DW), jnp.float32),
            pltpu.SemaphoreType.DMA,
        ],
    )
    def deg_kernel(dst_hbm, ones_hbm, zero_hbm, out_hbm, didx, ones_v, acc, sem):
        cid = lax.axis_index("c")
        sid = lax.axis_index("s")
        wid = cid * NS + sid
        pltpu.sync_copy(zero_hbm.at[pl.ds(sid * zrows, zrows)],
                        acc.at[pl.ds(sid * zrows, zrows)])
        pltpu.sync_copy(dst_hbm.at[wid], didx)
        pltpu.sync_copy(ones_hbm, ones_v)
        plsc.subcore_barrier()

        def fire(j, carry):
            pltpu.async_copy(ones_v, acc.at[didx.at[j]], sem, add=True)
            return carry

        lax.fori_loop(0, nc, fire, 0)

        def drain(j, carry):
            pltpu.make_async_copy(ones_v, acc.at[didx.at[j]], sem).wait()
            return carry

        lax.fori_loop(0, nc, drain, 0)
        plsc.subcore_barrier()
        pltpu.sync_copy(acc.at[pl.ds(sid * zrows, zrows)],
                        out_hbm.at[pl.ds(cid * n_pad + sid * zrows, zrows)])

    return deg_kernel


@functools.lru_cache(maxsize=None)
def _make_agg(n_pad, d, e_w):
    zrows = n_pad // NS
    nc = e_w // K
    assert nc % 2 == 0

    @functools.partial(
        pl.kernel,
        out_type=jax.ShapeDtypeStruct((NC * n_pad, d), jnp.float32),
        mesh=_sc_mesh(),
        compiler_params=pltpu.CompilerParams(use_tc_tiling_on_sc=False),
        scratch_types=[
            pltpu.VMEM((nc + 1, K), jnp.int32),
            pltpu.VMEM((nc, K), jnp.int32),
            pltpu.VMEM((K, d), jnp.float32),
            pltpu.VMEM((K, d), jnp.float32),
            pltpu.VMEM_SHARED((n_pad, d), jnp.float32),
            pltpu.SemaphoreType.DMA,
            pltpu.SemaphoreType.DMA,
        ],
    )
    def agg_kernel(src_hbm, dst_hbm, h_hbm, zero_hbm, out_hbm,
                   sidx, didx, rows0, rows1, acc, g0, g1):
        cid = lax.axis_index("c")
        sid = lax.axis_index("s")
        wid = cid * NS + sid
        pltpu.sync_copy(zero_hbm.at[pl.ds(sid * zrows, zrows)],
                        acc.at[pl.ds(sid * zrows, zrows)])
        pltpu.sync_copy(src_hbm.at[wid], sidx.at[pl.ds(0, nc)])
        pltpu.sync_copy(dst_hbm.at[wid], didx)
        pltpu.sync_copy(src_hbm.at[wid, 0], sidx.at[nc])
        plsc.subcore_barrier()

        pltpu.async_copy(h_hbm.at[sidx.at[0]], rows0, g0)

        def body(t, carry):
            j0 = 2 * t
            pltpu.async_copy(h_hbm.at[sidx.at[j0 + 1]], rows1, g1)
            pltpu.make_async_copy(h_hbm.at[sidx.at[j0]], rows0, g0).wait()
            pltpu.sync_copy(rows0, acc.at[didx.at[j0]], add=True)
            pltpu.async_copy(h_hbm.at[sidx.at[j0 + 2]], rows0, g0)
            pltpu.make_async_copy(h_hbm.at[sidx.at[j0 + 1]], rows1, g1).wait()
            pltpu.sync_copy(rows1, acc.at[didx.at[j0 + 1]], add=True)
            return carry

        lax.fori_loop(0, nc // 2, body, 0)
        pltpu.make_async_copy(h_hbm.at[sidx.at[nc]], rows0, g0).wait()
        plsc.subcore_barrier()
        pltpu.sync_copy(acc.at[pl.ds(sid * zrows, zrows)],
                        out_hbm.at[pl.ds(cid * n_pad + sid * zrows, zrows)])

    return agg_kernel


def _tc1_body(x_ref, w_ref, d0_ref, d1_ref, h_ref, dis_ref):
    deg = d0_ref[...] + d1_ref[...] + 1.0
    dis = lax.rsqrt(deg)
    dis_ref[...] = dis
    h_ref[...] = dis * jnp.dot(x_ref[...], w_ref[...],
                               preferred_element_type=jnp.float32)


def _tc2_body(a0_ref, a1_ref, h1s_ref, dis_ref, b1_ref, w2_ref, out_ref):
    dis = dis_ref[...]
    h = dis * (a0_ref[...] + a1_ref[...] + h1s_ref[...]) + b1_ref[...]
    h = jnp.maximum(h, 0.0)
    out_ref[...] = dis * jnp.dot(h, w2_ref[...],
                                 preferred_element_type=jnp.float32)


def _tc3_body(a0_ref, a1_ref, h2s_ref, dis_ref, b2_ref, out_ref):
    dis = dis_ref[...]
    t = dis * (a0_ref[...] + a1_ref[...] + h2s_ref[...]) + b2_ref[...]
    m = jnp.max(t, axis=1, keepdims=True)
    lse = jnp.log(jnp.sum(jnp.exp(t - m), axis=1, keepdims=True)) + m
    out_ref[...] = t - lse


def kernel(x, edge_index, W1, b1, W2, b2):
    N, d_in = x.shape
    d_h = W1.shape[1]
    d_out = W2.shape[1]
    E = edge_index.shape[1]
    f32 = jnp.float32

    e_w = -(-E // (NW * 2 * K)) * 2 * K
    e_pad = NW * e_w
    n_pad = -(-(N + 1) // 128) * 128
    pad = e_pad - E
    nc = e_w // K

    src_p = jnp.concatenate([edge_index[0],
                             jnp.zeros((pad,), edge_index.dtype)]
                            ).reshape(NW, nc, K)
    dst_p = jnp.concatenate([edge_index[1],
                             jnp.full((pad,), N, edge_index.dtype)]
                            ).reshape(NW, nc, K)

    degs = _make_deg(n_pad, e_w)(
        dst_p, jnp.ones((K, DW), f32), jnp.zeros((n_pad, DW), f32))
    d0 = degs[:N, :1]
    d1 = degs[n_pad:n_pad + N, :1]

    R = 2000
    grid = (N // R,)
    h1s, dis = pl.pallas_call(
        _tc1_body,
        grid=grid,
        in_specs=[
            pl.BlockSpec((R, d_in), lambda i: (i, 0)),
            pl.BlockSpec((d_in, d_h), lambda i: (0, 0)),
            pl.BlockSpec((R, 1), lambda i: (i, 0)),
            pl.BlockSpec((R, 1), lambda i: (i, 0)),
        ],
        out_specs=[
            pl.BlockSpec((R, d_h), lambda i: (i, 0)),
            pl.BlockSpec((R, 1), lambda i: (i, 0)),
        ],
        out_shape=[
            jax.ShapeDtypeStruct((N, d_h), f32),
            jax.ShapeDtypeStruct((N, 1), f32),
        ],
    )(x, W1, d0, d1)

    agg1 = _make_agg(n_pad, d_h, e_w)(
        src_p, dst_p, h1s, jnp.zeros((n_pad, d_h), f32))

    h2s = pl.pallas_call(
        _tc2_body,
        grid=grid,
        in_specs=[
            pl.BlockSpec((R, d_h), lambda i: (i, 0)),
            pl.BlockSpec((R, d_h), lambda i: (i, 0)),
            pl.BlockSpec((R, d_h), lambda i: (i, 0)),
            pl.BlockSpec((R, 1), lambda i: (i, 0)),
            pl.BlockSpec((1, d_h), lambda i: (0, 0)),
            pl.BlockSpec((d_h, d_out), lambda i: (0, 0)),
        ],
        out_specs=pl.BlockSpec((R, d_out), lambda i: (i, 0)),
        out_shape=jax.ShapeDtypeStruct((N, d_out), f32),
    )(agg1[:N], agg1[n_pad:n_pad + N], h1s, dis,
      b1.reshape(1, d_h), W2)

    agg2 = _make_agg(n_pad, d_out, e_w)(
        src_p, dst_p, h2s, jnp.zeros((n_pad, d_out), f32))

    out = pl.pallas_call(
        _tc3_body,
        grid=grid,
        in_specs=[
            pl.BlockSpec((R, d_out), lambda i: (i, 0)),
            pl.BlockSpec((R, d_out), lambda i: (i, 0)),
            pl.BlockSpec((R, d_out), lambda i: (i, 0)),
            pl.BlockSpec((R, 1), lambda i: (i, 0)),
            pl.BlockSpec((1, d_out), lambda i: (0, 0)),
        ],
        out_specs=pl.BlockSpec((R, d_out), lambda i: (i, 0)),
        out_shape=jax.ShapeDtypeStruct((N, d_out), f32),
    )(agg2[:N], agg2[n_pad:n_pad + N], h2s, dis, b2.reshape(1, d_out))

    return out
